# Initial kernel scaffold; baseline (speedup 1.0000x reference)
import jax
import jax.numpy as jnp
from jax.experimental import pallas as pl


def kernel(x, edge_index, edge_type, comp0, bases0, root0, bias0, comp1, bases1, root1, bias1, comp2, bases2, root2, bias2):
    raise NotImplementedError("write your pallas kernel here")


# speedup vs baseline: 7.6435x; 7.6435x over previous
import functools
import jax
import jax.numpy as jnp
from jax import lax
from jax.experimental import pallas as pl
from jax.experimental.pallas import tpu as pltpu
from jax.experimental.pallas import tpu_sc as plsc

NC = 2
NS = 16
NW = NC * NS
CK = 128


def _full16(v):
  return jnp.full((16,), v, jnp.int32)


def _sgather_kernel(invc_hbm, key_hbm, out_hbm, invc_v, key_v, s_v, nchunks):
  cid = lax.axis_index("c")
  sid = lax.axis_index("s")
  wid = sid * NC + cid
  pltpu.sync_copy(invc_hbm, invc_v)
  pltpu.sync_copy(key_hbm.at[wid], key_v)

  def body(j, _):
    for q in range(CK // 16):
      kv = key_v[j, pl.ds(q * 16, 16)]
      sv = plsc.load_gather(invc_v, [kv >> 7, kv & 127])
      s_v[j, pl.ds(q * 16, 16)] = sv
    return 0

  lax.fori_loop(0, nchunks, body, 0)
  pltpu.sync_copy(s_v, out_hbm.at[wid])


def _agg_kernel(y_hbm, gidx_hbm, dst_hbm, s_hbm, zeros_hbm, out_hbm,
                gidx_v, dst_v, s_v, rows_v, acc, sem, nchunks, arows):
  cid = lax.axis_index("c")
  sid = lax.axis_index("s")
  wid = sid * NC + cid
  rows_per_tile = arows // NS
  pltpu.sync_copy(zeros_hbm.at[pl.ds(sid * rows_per_tile, rows_per_tile)],
                  acc.at[pl.ds(sid * rows_per_tile, rows_per_tile)])
  pltpu.sync_copy(gidx_hbm.at[wid], gidx_v)
  pltpu.sync_copy(dst_hbm.at[wid], dst_v)
  pltpu.sync_copy(s_hbm.at[wid], s_v)
  plsc.subcore_barrier()

  def chunk(j, _):
    pltpu.async_copy(y_hbm.at[gidx_v.at[j]], rows_v, sem).wait()

    def edge16(q, _):
      sv16 = s_v[j, pl.ds(q * 16, 16)]
      for k16 in range(16):
        kk = q * 16 + k16
        sval = sv16[k16]
        for h in range(8):
          rows_v[kk, pl.ds(h * 16, 16)] = rows_v[kk, pl.ds(h * 16, 16)] * sval
      return 0

    lax.fori_loop(0, CK // 16, edge16, 0)
    pltpu.sync_copy(rows_v, acc.at[dst_v.at[j]], add=True)
    return 0

  lax.fori_loop(0, nchunks, chunk, 0)
  plsc.subcore_barrier()
  pltpu.sync_copy(acc.at[pl.ds(sid * rows_per_tile, rows_per_tile)],
                  out_hbm.at[cid, pl.ds(sid * rows_per_tile, rows_per_tile)])


def _sc_sgather(invc, key_w, nchunks):
  mesh = plsc.VectorSubcoreMesh(core_axis_name="c", subcore_axis_name="s")
  kfn = functools.partial(
      pl.kernel,
      functools.partial(_sgather_kernel, nchunks=nchunks),
      mesh=mesh,
      out_type=jax.ShapeDtypeStruct((NW, nchunks, CK), jnp.float32),
      scratch_types=[
          pltpu.VMEM(invc.shape, jnp.float32),
          pltpu.VMEM((nchunks, CK), jnp.int32),
          pltpu.VMEM((nchunks, CK), jnp.float32),
      ],
      compiler_params=pltpu.CompilerParams(needs_layout_passes=False),
  )()
  return kfn(invc, key_w)


def _sc_aggregate(y3, gidx_w, dst_w, s_w, arows, nchunks):
  mesh = plsc.VectorSubcoreMesh(core_axis_name="c", subcore_axis_name="s")
  kfn = functools.partial(
      pl.kernel,
      functools.partial(_agg_kernel, nchunks=nchunks, arows=arows),
      mesh=mesh,
      out_type=jax.ShapeDtypeStruct((NC, arows, 128), jnp.float32),
      scratch_types=[
          pltpu.VMEM((nchunks, CK), jnp.int32),
          pltpu.VMEM((nchunks, CK), jnp.int32),
          pltpu.VMEM((nchunks, CK), jnp.float32),
          pltpu.VMEM((CK, 128), jnp.float32),
          pltpu.VMEM_SHARED((arows, 128), jnp.float32),
          pltpu.SemaphoreType.DMA,
      ],
      compiler_params=pltpu.CompilerParams(needs_layout_passes=False),
  )()
  zeros = jnp.zeros((arows, 128), jnp.float32)
  return kfn(y3, gidx_w, dst_w, s_w, zeros)


def _dense_body(ws_ref, b0_ref, b1_ref, rt_ref, x_ref, o_ref):
  rr = pl.program_id(0)
  w = (ws_ref[rr, 0] * b0_ref[...] + ws_ref[rr, 1] * b1_ref[...]
       + ws_ref[rr, 2] * rt_ref[...])
  o_ref[...] = jnp.dot(x_ref[...], w, preferred_element_type=jnp.float32)


def _tc_dense(x, wsel, b0, b1, rt, bn):
  n, h = x.shape
  nb = n // bn
  return pl.pallas_call(
      _dense_body,
      grid=(9, nb),
      in_specs=[
          pl.BlockSpec(memory_space=pltpu.SMEM),
          pl.BlockSpec((h, h), lambda r, i: (0, 0)),
          pl.BlockSpec((h, h), lambda r, i: (0, 0)),
          pl.BlockSpec((h, h), lambda r, i: (0, 0)),
          pl.BlockSpec((bn, h), lambda r, i: (i, 0)),
      ],
      out_specs=pl.BlockSpec((bn, h), lambda r, i: (r * nb + i, 0)),
      out_shape=jax.ShapeDtypeStruct((9 * n, h), jnp.float32),
  )(wsel, b0, b1, rt, x)


def _invc_body(p0_ref, p1_ref, o_ref, nreal):
  c = p0_ref[...] + p1_ref[...]
  idx = jax.lax.broadcasted_iota(jnp.int32, c.shape, 0) * 128 + \
      jax.lax.broadcasted_iota(jnp.int32, c.shape, 1)
  inv = 1.0 / jnp.maximum(c, 1.0)
  o_ref[...] = jnp.where(idx < nreal, inv, 0.0)


def _tc_invc(p0, p1, nreal):
  rows = p0.shape[0]
  return pl.pallas_call(
      functools.partial(_invc_body, nreal=nreal),
      grid=(1,),
      in_specs=[
          pl.BlockSpec((rows, 128), lambda i: (0, 0)),
          pl.BlockSpec((rows, 128), lambda i: (0, 0)),
      ],
      out_specs=pl.BlockSpec((rows, 128), lambda i: (0, 0)),
      out_shape=jax.ShapeDtypeStruct((rows, 128), jnp.float32),
  )(p0, p1)


def _combine_body(y_ref, p0_ref, p1_ref, b_ref, o_ref):
  o_ref[...] = y_ref[...] + p0_ref[...] + p1_ref[...] + b_ref[...]


def _tc_combine(yroot, p0, p1, bias, bn):
  n, h = yroot.shape
  nb = n // bn
  return pl.pallas_call(
      _combine_body,
      grid=(nb,),
      in_specs=[
          pl.BlockSpec((bn, h), lambda i: (i, 0)),
          pl.BlockSpec((bn, h), lambda i: (i, 0)),
          pl.BlockSpec((bn, h), lambda i: (i, 0)),
          pl.BlockSpec((1, h), lambda i: (0, 0)),
      ],
      out_specs=pl.BlockSpec((bn, h), lambda i: (i, 0)),
      out_shape=jax.ShapeDtypeStruct((n, h), jnp.float32),
  )(yroot, p0, p1, bias)


def _pad_w(a, w):
  return jnp.pad(a, [(0, 0)] * (a.ndim - 1) + [(0, w - a.shape[-1])])


def _layer(x, gidx_w, dst_w, s_w, wsel, b0, b1, rt, bias, n, arows, nchunks):
  h = x.shape[1]
  y = _tc_dense(x, wsel, b0, b1, rt, 1000)
  parts = _sc_aggregate(y, gidx_w, dst_w, s_w, arows, nchunks)
  parts = parts[:, :n]
  yroot = y[8 * n:]
  bias_row = _pad_w(bias, h).reshape(1, h)
  return _tc_combine(yroot, parts[0], parts[1], bias_row, 1000)


def kernel(x, edge_index, edge_type, comp0, bases0, root0, bias0,
           comp1, bases1, root1, bias1, comp2, bases2, root2, bias2):
  n, h = x.shape
  e = edge_index.shape[1]
  r = comp0.shape[0]

  src = edge_index[0]
  dst = edge_index[1]
  et = edge_type

  nchunks = -(-e // (NW * CK))
  e_pad = NW * CK * nchunks
  crows = ((n * r + 127) // 128) * 128 + 128
  arows = ((n + NS * 8 - 1) // (NS * 8)) * (NS * 8)

  gidx = et * n + src
  key = dst * r + et
  pad = e_pad - e
  gidx_w = jnp.concatenate([gidx, jnp.zeros((pad,), jnp.int32)]) \
      .reshape(NW, nchunks, CK)
  dst_w = jnp.concatenate([dst, jnp.zeros((pad,), jnp.int32)]) \
      .reshape(NW, nchunks, CK)
  et_w = jnp.concatenate([et, jnp.full((pad,), r, jnp.int32)]) \
      .reshape(NW, nchunks, CK)
  key_w = jnp.concatenate([key, jnp.full((pad,), n * r, jnp.int32)]) \
      .reshape(NW, nchunks, CK)

  eye = jnp.concatenate([jnp.eye(r, 128, dtype=jnp.float32),
                         jnp.zeros((16 - r, 128), jnp.float32)])
  ones_w = jnp.ones((NW, nchunks, CK), jnp.float32)
  cparts = _sc_aggregate(eye, et_w, dst_w, ones_w, arows, nchunks)
  cp = cparts[:, :n, :r].reshape(NC, n * r)
  cp = jnp.pad(cp, ((0, 0), (0, crows - n * r))).reshape(NC, crows // 128, 128)
  invc = _tc_invc(cp[0], cp[1], n * r)
  s_w = _sc_sgather(invc, key_w, nchunks)

  def wsel_of(comp):
    return jnp.concatenate([
        jnp.concatenate([comp, jnp.zeros((r, 2), comp.dtype)], axis=1),
        jnp.array([[0.0, 0.0, 1.0, 0.0]], comp.dtype)], axis=0)

  h1 = _layer(x, gidx_w, dst_w, s_w, wsel_of(comp0), bases0[0], bases0[1],
              root0, bias0, n, arows, nchunks)
  h2 = _layer(h1, gidx_w, dst_w, s_w, wsel_of(comp1), bases1[0], bases1[1],
              root1, bias1, n, arows, nchunks)
  h3 = _layer(h2, gidx_w, dst_w, s_w, wsel_of(comp2),
              _pad_w(bases2[0], h), _pad_w(bases2[1], h),
              _pad_w(root2, h), bias2, n, arows, nchunks)
  return h3[:, :1]
